# baseline (speedup 1.0000x reference)
import jax
import jax.numpy as jnp
from jax.experimental import pallas as pl
from jax.experimental.pallas import tpu as pltpu

_KK = 4
_CHUNK = 512


def _pool_kernel(x_ref, p_ref, o_ref):
    groups = p_ref.shape[1]
    for j in range(x_ref.shape[1] // _CHUNK):
        o_ref[:, j * groups:(j + 1) * groups] = jnp.dot(
            x_ref[:, j * _CHUNK:(j + 1) * _CHUNK].astype(jnp.bfloat16),
            p_ref[...],
            preferred_element_type=jnp.float32,
        ).astype(o_ref.dtype)


def kernel(x):
    B, C = x.shape
    ncout = C // _KK
    k = float(C) / float(ncout)
    scale = (k ** 0.5) / float(_KK)

    groups = _CHUNK // _KK
    rows = jnp.arange(_CHUNK, dtype=jnp.int32)[:, None]
    cols = jnp.arange(groups, dtype=jnp.int32)[None, :]
    p = jnp.where(rows // _KK == cols, scale, 0.0).astype(jnp.bfloat16)

    tb = 256
    grid = (B // tb,)
    return pl.pallas_call(
        _pool_kernel,
        grid=grid,
        in_specs=[
            pl.BlockSpec((tb, C), lambda i: (i, 0)),
            pl.BlockSpec((_CHUNK, groups), lambda i: (0, 0)),
        ],
        out_specs=pl.BlockSpec((tb, ncout), lambda i: (i, 0)),
        out_shape=jax.ShapeDtypeStruct((B, ncout), x.dtype),
        compiler_params=pltpu.CompilerParams(
            dimension_semantics=("parallel",),
            vmem_limit_bytes=32 * 1024 * 1024,
        ),
    )(x, p)


# speedup vs baseline: 1.4026x; 1.4026x over previous
import jax
import jax.numpy as jnp
from jax.experimental import pallas as pl
from jax.experimental.pallas import tpu as pltpu

_KK = 4
_CHUNK = 512


def _pool_kernel(x_ref, p_ref, o_ref):
    groups = p_ref.shape[1]
    for j in range(x_ref.shape[1] // _CHUNK):
        o_ref[:, j * groups:(j + 1) * groups] = jnp.dot(
            x_ref[:, j * _CHUNK:(j + 1) * _CHUNK].astype(jnp.bfloat16),
            p_ref[...],
            preferred_element_type=jnp.float32,
        ).astype(o_ref.dtype)


def kernel(x):
    B, C = x.shape
    ncout = C // _KK
    k = float(C) / float(ncout)
    scale = (k ** 0.5) / float(_KK)

    groups = _CHUNK // _KK
    rows = jnp.arange(_CHUNK, dtype=jnp.int32)[:, None]
    cols = jnp.arange(groups, dtype=jnp.int32)[None, :]
    p = jnp.where(rows // _KK == cols, scale, 0.0).astype(jnp.bfloat16)

    tb = 1024
    grid = (B // tb,)
    return pl.pallas_call(
        _pool_kernel,
        grid=grid,
        in_specs=[
            pl.BlockSpec((tb, C), lambda i: (i, 0)),
            pl.BlockSpec((_CHUNK, groups), lambda i: (0, 0)),
        ],
        out_specs=pl.BlockSpec((tb, ncout), lambda i: (i, 0)),
        out_shape=jax.ShapeDtypeStruct((B, ncout), x.dtype),
        compiler_params=pltpu.CompilerParams(
            dimension_semantics=("parallel",),
            vmem_limit_bytes=32 * 1024 * 1024,
        ),
    )(x, p)
